# baseline (speedup 1.0000x reference)
import functools

import jax
import jax.numpy as jnp
from jax import lax
from jax.experimental import pallas as pl
from jax.experimental.pallas import tpu as pltpu
from jax.experimental.pallas import tpu_sc as plsc

N_NODES = 100000
T = 2
U = 16
D = 128
B = 10000
E = 1600000

NC = 2
NS = 16
NW = NC * NS

RC = 16
CHUNKS = 25
RPW = RC * CHUNKS
EPW = RPW * 128
E_PAD = NW * EPW
ROWS_T = E_PAD // 128

B_ACC = 10240
ROWS_B = B_ACC // NS
GPW = 3
BG = NW * GPW * 128


def _sc_kernel(src_hbm, dst_hbm, nte_hbm, onodes_hbm, nemb_hbm,
               partial_hbm, gathered_hbm,
               src_v, dst_v, gidx_v, rows_v, zbuf_v, oidx_v, grows_v,
               acc0, acc1, sem):
    cid = lax.axis_index("c")
    sid = lax.axis_index("s")
    w = cid * NS + sid

    zero = jnp.zeros((16,), jnp.float32)

    def zbody(i, _):
        zbuf_v[i, :] = zero
        return 0

    lax.fori_loop(0, ROWS_B, zbody, 0)
    pltpu.sync_copy(zbuf_v, acc0.at[pl.ds(sid * ROWS_B, ROWS_B)])
    pltpu.sync_copy(zbuf_v, acc1.at[pl.ds(sid * ROWS_B, ROWS_B)])
    plsc.subcore_barrier()

    for t in range(T):
        acc = acc0 if t == 0 else acc1

        def cbody(c, _, t=t, acc=acc):
            base = w * RPW + c * RC
            pltpu.sync_copy(src_hbm.at[t, pl.ds(base, RC)], src_v)
            pltpu.sync_copy(dst_hbm.at[t, pl.ds(base, RC)], dst_v)

            def ibody(j, _):
                for l in range(8):
                    gidx_v[j, pl.ds(l * 16, 16)] = (
                        src_v[j, pl.ds(l * 16, 16)] * T + t)
                return 0

            lax.fori_loop(0, RC, ibody, 0)

            def gfire(j, _):
                pltpu.make_async_copy(
                    nte_hbm.at[gidx_v.at[j]], rows_v.at[j], sem).start()
                return 0

            lax.fori_loop(0, RC, gfire, 0)

            def gdrain(j, _):
                pltpu.make_async_copy(
                    nte_hbm.at[gidx_v.at[j]], rows_v.at[j], sem).wait()
                return 0

            lax.fori_loop(0, RC, gdrain, 0)

            def sbody(j, _):
                pltpu.sync_copy(rows_v.at[j], acc.at[dst_v.at[j]], add=True)
                return 0

            lax.fori_loop(0, RC, sbody, 0)
            return 0

        lax.fori_loop(0, CHUNKS, cbody, 0)

    plsc.subcore_barrier()

    pltpu.sync_copy(acc0.at[pl.ds(sid * ROWS_B, ROWS_B)],
                    partial_hbm.at[cid, 0, pl.ds(sid * ROWS_B, ROWS_B)])
    pltpu.sync_copy(acc1.at[pl.ds(sid * ROWS_B, ROWS_B)],
                    partial_hbm.at[cid, 1, pl.ds(sid * ROWS_B, ROWS_B)])

    pltpu.sync_copy(onodes_hbm.at[w], oidx_v)
    for s3 in range(GPW):
        r = w * GPW + s3
        pltpu.async_copy(nemb_hbm.at[oidx_v.at[s3]], grows_v, sem).wait()
        pltpu.sync_copy(grows_v, gathered_hbm.at[pl.ds(r * 128, 128)])


def _tc_body(partial_ref, gathered_ref, s1_ref, s2_ref, w_ref, out_ref):
    p = partial_ref[...]
    agg0 = p[0, 0] + p[1, 0]
    agg1 = p[0, 1] + p[1, 1]
    s1 = s1_ref[...]
    s2 = s2_ref[...]
    h0 = jnp.tanh(jnp.dot(agg0, s1[0], preferred_element_type=jnp.float32))
    h1 = jnp.tanh(jnp.dot(agg1, s1[1], preferred_element_type=jnp.float32))
    l0 = jnp.sum(h0 * s2[0][None, :], axis=1, keepdims=True)
    l1 = jnp.sum(h1 * s2[1][None, :], axis=1, keepdims=True)
    m = jnp.maximum(l0, l1)
    e0 = jnp.exp(l0 - m)
    e1 = jnp.exp(l1 - m)
    inv = 1.0 / (e0 + e1)
    mixed = (e0 * inv) * agg0 + (e1 * inv) * agg1
    g = gathered_ref[...]
    wts = w_ref[...]
    for t in range(T):
        v = g + jnp.dot(mixed, wts[t], preferred_element_type=jnp.float32)
        nrm = jnp.sqrt(jnp.sum(v * v, axis=1, keepdims=True))
        out_ref[:, t, :] = v / jnp.maximum(nrm, 1e-12)


def kernel(node_embeddings, node_type_embeddings, trans_weights,
           trans_weights_s1, trans_weights_s2, input_nodes, output_nodes,
           edge_src, edge_dst):
    nte = node_type_embeddings.reshape(N_NODES * T, U)

    pad = E_PAD - E
    src_p = jnp.concatenate(
        [edge_src, jnp.zeros((T, pad), jnp.int32)], axis=1
    ).reshape(T, ROWS_T, 128)
    pad_dst = B + (jnp.arange(pad, dtype=jnp.int32) % 128)
    dst_p = jnp.concatenate(
        [edge_dst, jnp.broadcast_to(pad_dst, (T, pad))], axis=1
    ).reshape(T, ROWS_T, 128)
    onodes = jnp.concatenate(
        [output_nodes, jnp.zeros((BG - B,), jnp.int32)]
    ).reshape(NW, GPW, 128)

    sc = functools.partial(
        pl.kernel,
        out_type=[
            jax.ShapeDtypeStruct((NC, T, B_ACC, U), jnp.float32),
            jax.ShapeDtypeStruct((BG, D), jnp.float32),
        ],
        mesh=plsc.VectorSubcoreMesh(core_axis_name="c", subcore_axis_name="s"),
        compiler_params=pltpu.CompilerParams(use_tc_tiling_on_sc=False),
        scratch_types=[
            pltpu.VMEM((RC, 128), jnp.int32),
            pltpu.VMEM((RC, 128), jnp.int32),
            pltpu.VMEM((RC, 128), jnp.int32),
            pltpu.VMEM((RC, 128, U), jnp.float32),
            pltpu.VMEM((ROWS_B, U), jnp.float32),
            pltpu.VMEM((GPW, 128), jnp.int32),
            pltpu.VMEM((128, D), jnp.float32),
            pltpu.VMEM_SHARED((B_ACC, U), jnp.float32),
            pltpu.VMEM_SHARED((B_ACC, U), jnp.float32),
            pltpu.SemaphoreType.DMA,
        ],
    )(_sc_kernel)
    partial, gathered = sc(src_p, dst_p, nte, onodes, node_embeddings)

    BB = 2000
    s2r = trans_weights_s2.reshape(T, 16)
    out = pl.pallas_call(
        _tc_body,
        grid=(B // BB,),
        in_specs=[
            pl.BlockSpec((NC, T, BB, U), lambda i: (0, 0, i, 0)),
            pl.BlockSpec((BB, D), lambda i: (i, 0)),
            pl.BlockSpec((T, U, 16), lambda i: (0, 0, 0)),
            pl.BlockSpec((T, 16), lambda i: (0, 0)),
            pl.BlockSpec((T, U, D), lambda i: (0, 0, 0)),
        ],
        out_specs=pl.BlockSpec((BB, T, D), lambda i: (i, 0, 0)),
        out_shape=jax.ShapeDtypeStruct((B, T, D), jnp.float32),
    )(partial, gathered, trans_weights_s1, s2r, trans_weights)
    return out


# speedup vs baseline: 1.6107x; 1.6107x over previous
import functools

import jax
import jax.numpy as jnp
from jax import lax
from jax.experimental import pallas as pl
from jax.experimental.pallas import tpu as pltpu
from jax.experimental.pallas import tpu_sc as plsc

N_NODES = 100000
T = 2
U = 16
D = 128
B = 10000
E = 1600000

NC = 2
NS = 16
NW = NC * NS

RC = 16
CHUNKS = 25
RPW = RC * CHUNKS
EPW = RPW * 128
E_PAD = NW * EPW
ROWS_T = E_PAD // 128

B_ACC = 10240
ROWS_B = B_ACC // NS
GPW = 3
BG = NW * GPW * 128


def _sc_kernel(src_hbm, dst_hbm, nte_hbm, onodes_hbm, nemb_hbm,
               partial_hbm, gathered_hbm,
               src_v, dst_v, gidx_v, rows_v, zbuf_v, oidx_v, grows_v,
               acc0, acc1, sem):
    cid = lax.axis_index("c")
    sid = lax.axis_index("s")
    w = cid * NS + sid

    zero = jnp.zeros((16,), jnp.float32)

    def zbody(i, _):
        zbuf_v[i, :] = zero
        return 0

    lax.fori_loop(0, ROWS_B, zbody, 0)
    pltpu.sync_copy(zbuf_v, acc0.at[pl.ds(sid * ROWS_B, ROWS_B)])
    pltpu.sync_copy(zbuf_v, acc1.at[pl.ds(sid * ROWS_B, ROWS_B)])
    plsc.subcore_barrier()

    for t in range(T):
        acc = acc0 if t == 0 else acc1

        def cbody(c, _, t=t, acc=acc):
            base = w * RPW + c * RC
            pltpu.sync_copy(src_hbm.at[t, pl.ds(base, RC)], src_v)
            pltpu.sync_copy(dst_hbm.at[t, pl.ds(base, RC)], dst_v)

            def ibody(j, _):
                for l in range(8):
                    gidx_v[j, pl.ds(l * 16, 16)] = (
                        src_v[j, pl.ds(l * 16, 16)] * T + t)
                return 0

            lax.fori_loop(0, RC, ibody, 0)

            def gfire(j, _):
                pltpu.make_async_copy(
                    nte_hbm.at[gidx_v.at[j]], rows_v.at[j], sem).start()
                return 0

            lax.fori_loop(0, RC, gfire, 0)

            def gdrain(j, _):
                pltpu.make_async_copy(
                    nte_hbm.at[gidx_v.at[j]], rows_v.at[j], sem).wait()
                return 0

            lax.fori_loop(0, RC, gdrain, 0)

            def sbody(j, _):
                pltpu.sync_copy(rows_v.at[j], acc.at[dst_v.at[j]], add=True)
                return 0

            lax.fori_loop(0, RC, sbody, 0)
            return 0

        lax.fori_loop(0, CHUNKS, cbody, 0)

    plsc.subcore_barrier()

    pltpu.sync_copy(acc0.at[pl.ds(sid * ROWS_B, ROWS_B)],
                    partial_hbm.at[cid, 0, pl.ds(sid * ROWS_B, ROWS_B)])
    pltpu.sync_copy(acc1.at[pl.ds(sid * ROWS_B, ROWS_B)],
                    partial_hbm.at[cid, 1, pl.ds(sid * ROWS_B, ROWS_B)])

    pltpu.sync_copy(onodes_hbm.at[w], oidx_v)
    for s3 in range(GPW):
        r = w * GPW + s3
        pltpu.async_copy(nemb_hbm.at[oidx_v.at[s3]], grows_v, sem).wait()
        pltpu.sync_copy(grows_v, gathered_hbm.at[pl.ds(r * 128, 128)])


def _tc_body(partial_ref, gathered_ref, s1_ref, s2_ref, w_ref, out_ref):
    p = partial_ref[...]
    agg0 = p[0, 0] + p[1, 0]
    agg1 = p[0, 1] + p[1, 1]
    s1 = s1_ref[...]
    s2 = s2_ref[...]
    h0 = jnp.tanh(jnp.dot(agg0, s1[0], preferred_element_type=jnp.float32))
    h1 = jnp.tanh(jnp.dot(agg1, s1[1], preferred_element_type=jnp.float32))
    l0 = jnp.sum(h0 * s2[0][None, :], axis=1, keepdims=True)
    l1 = jnp.sum(h1 * s2[1][None, :], axis=1, keepdims=True)
    m = jnp.maximum(l0, l1)
    e0 = jnp.exp(l0 - m)
    e1 = jnp.exp(l1 - m)
    inv = 1.0 / (e0 + e1)
    mixed = (e0 * inv) * agg0 + (e1 * inv) * agg1
    g = gathered_ref[...]
    wts = w_ref[...]
    for t in range(T):
        v = g + jnp.dot(mixed, wts[t], preferred_element_type=jnp.float32)
        nrm = jnp.sqrt(jnp.sum(v * v, axis=1, keepdims=True))
        out_ref[:, t, :] = v / jnp.maximum(nrm, 1e-12)


def kernel(node_embeddings, node_type_embeddings, trans_weights,
           trans_weights_s1, trans_weights_s2, input_nodes, output_nodes,
           edge_src, edge_dst):
    nte = node_type_embeddings.reshape(N_NODES * T, U)

    pad = E_PAD - E
    pad_src = jnp.arange(pad, dtype=jnp.int32) % 128
    src_p = jnp.concatenate(
        [edge_src, jnp.broadcast_to(pad_src, (T, pad))], axis=1
    ).reshape(T, ROWS_T, 128)
    pad_dst = B + (jnp.arange(pad, dtype=jnp.int32) % 128)
    dst_p = jnp.concatenate(
        [edge_dst, jnp.broadcast_to(pad_dst, (T, pad))], axis=1
    ).reshape(T, ROWS_T, 128)
    onodes = jnp.concatenate(
        [output_nodes, jnp.zeros((BG - B,), jnp.int32)]
    ).reshape(NW, GPW, 128)

    sc = functools.partial(
        pl.kernel,
        out_type=[
            jax.ShapeDtypeStruct((NC, T, B_ACC, U), jnp.float32),
            jax.ShapeDtypeStruct((BG, D), jnp.float32),
        ],
        mesh=plsc.VectorSubcoreMesh(core_axis_name="c", subcore_axis_name="s"),
        compiler_params=pltpu.CompilerParams(use_tc_tiling_on_sc=False),
        scratch_types=[
            pltpu.VMEM((RC, 128), jnp.int32),
            pltpu.VMEM((RC, 128), jnp.int32),
            pltpu.VMEM((RC, 128), jnp.int32),
            pltpu.VMEM((RC, 128, U), jnp.float32),
            pltpu.VMEM((ROWS_B, U), jnp.float32),
            pltpu.VMEM((GPW, 128), jnp.int32),
            pltpu.VMEM((128, D), jnp.float32),
            pltpu.VMEM_SHARED((B_ACC, U), jnp.float32),
            pltpu.VMEM_SHARED((B_ACC, U), jnp.float32),
            pltpu.SemaphoreType.DMA,
        ],
    )(_sc_kernel)
    partial, gathered = sc(src_p, dst_p, nte, onodes, node_embeddings)

    BB = 2000
    s2r = trans_weights_s2.reshape(T, 16)
    out = pl.pallas_call(
        _tc_body,
        grid=(B // BB,),
        in_specs=[
            pl.BlockSpec((NC, T, BB, U), lambda i: (0, 0, i, 0)),
            pl.BlockSpec((BB, D), lambda i: (i, 0)),
            pl.BlockSpec((T, U, 16), lambda i: (0, 0, 0)),
            pl.BlockSpec((T, 16), lambda i: (0, 0)),
            pl.BlockSpec((T, U, D), lambda i: (0, 0, 0)),
        ],
        out_specs=pl.BlockSpec((BB, T, D), lambda i: (i, 0, 0)),
        out_shape=jax.ShapeDtypeStruct((B, T, D), jnp.float32),
    )(partial, gathered, trans_weights_s1, s2r, trans_weights)
    return out


# speedup vs baseline: 1.8205x; 1.1303x over previous
import functools

import jax
import jax.numpy as jnp
from jax import lax
from jax.experimental import pallas as pl
from jax.experimental.pallas import tpu as pltpu
from jax.experimental.pallas import tpu_sc as plsc

N_NODES = 100000
T = 2
U = 16
D = 128
B = 10000
E = 1600000

NC = 2
NS = 16
NW = NC * NS

ROWS_T = E // 128
RC = 16
FULL_CHUNKS = 781
C0PW = 27
C0 = 16 * C0PW
C1SPLIT = 13
C1A = 22
C1B = 21
TAIL_BASE = FULL_CHUNKS * RC
TAIL_R = 4

B_ACC = 10112
ROWS_B = B_ACC // NS
G_FULL = B // 128
G_ROWS = G_FULL + 1
BG = G_ROWS * 128


def _sc_kernel(src_hbm, dst_hbm, nte_hbm, onodes_hbm, nemb_hbm,
               partial_hbm, gathered_hbm,
               src_v, dst_v, gidx_v, rows_v, zbuf_v, oidx_v, grows_v,
               acc0, acc1, sem):
    cid = lax.axis_index("c")
    sid = lax.axis_index("s")
    w = cid * NS + sid

    zero = jnp.zeros((16,), jnp.float32)

    def zbody(i, _):
        zbuf_v[i, :] = zero
        return 0

    lax.fori_loop(0, ROWS_B, zbody, 0)
    pltpu.sync_copy(zbuf_v, acc0.at[pl.ds(sid * ROWS_B, ROWS_B)])
    pltpu.sync_copy(zbuf_v, acc1.at[pl.ds(sid * ROWS_B, ROWS_B)])
    plsc.subcore_barrier()

    nchunks = jnp.where(cid == 0, C0PW,
                        jnp.where(sid < C1SPLIT, C1A, C1B))
    chunk0 = jnp.where(
        cid == 0, sid * C0PW,
        C0 + jnp.where(sid < C1SPLIT, sid * C1A,
                       C1SPLIT * C1A + (sid - C1SPLIT) * C1B))

    for t in range(T):
        acc = acc0 if t == 0 else acc1

        def cbody(c, _, t=t, acc=acc):
            base = (chunk0 + c) * RC
            pltpu.sync_copy(src_hbm.at[t, pl.ds(base, RC)], src_v)
            pltpu.sync_copy(dst_hbm.at[t, pl.ds(base, RC)], dst_v)

            def ibody(j, _):
                for l in range(8):
                    gidx_v[j, pl.ds(l * 16, 16)] = (
                        src_v[j, pl.ds(l * 16, 16)] * T + t)
                return 0

            lax.fori_loop(0, RC, ibody, 0)

            def gfire(j, _):
                pltpu.make_async_copy(
                    nte_hbm.at[gidx_v.at[j]], rows_v.at[j], sem).start()
                return 0

            lax.fori_loop(0, RC, gfire, 0)

            def gdrain(j, _):
                pltpu.make_async_copy(
                    nte_hbm.at[gidx_v.at[j]], rows_v.at[j], sem).wait()
                return 0

            lax.fori_loop(0, RC, gdrain, 0)

            def sbody(j, _):
                pltpu.sync_copy(rows_v.at[j], acc.at[dst_v.at[j]], add=True)
                return 0

            lax.fori_loop(0, RC, sbody, 0)
            return 0

        lax.fori_loop(0, nchunks, cbody, 0)

        def tbody(_, __, t=t, acc=acc):
            pltpu.sync_copy(src_hbm.at[t, pl.ds(TAIL_BASE, TAIL_R)],
                            src_v.at[pl.ds(0, TAIL_R)])
            pltpu.sync_copy(dst_hbm.at[t, pl.ds(TAIL_BASE, TAIL_R)],
                            dst_v.at[pl.ds(0, TAIL_R)])

            def tibody(j, _):
                for l in range(8):
                    gidx_v[j, pl.ds(l * 16, 16)] = (
                        src_v[j, pl.ds(l * 16, 16)] * T + t)
                return 0

            lax.fori_loop(0, TAIL_R, tibody, 0)

            def tgather(j, _):
                pltpu.make_async_copy(
                    nte_hbm.at[gidx_v.at[j]], rows_v.at[j], sem).start()
                pltpu.make_async_copy(
                    nte_hbm.at[gidx_v.at[j]], rows_v.at[j], sem).wait()
                pltpu.sync_copy(rows_v.at[j], acc.at[dst_v.at[j]], add=True)
                return 0

            lax.fori_loop(0, TAIL_R, tgather, 0)
            return 0

        lax.fori_loop(0, jnp.where(w == 0, 1, 0), tbody, 0)

    plsc.subcore_barrier()

    pltpu.sync_copy(acc0.at[pl.ds(sid * ROWS_B, ROWS_B)],
                    partial_hbm.at[cid, 0, pl.ds(sid * ROWS_B, ROWS_B)])
    pltpu.sync_copy(acc1.at[pl.ds(sid * ROWS_B, ROWS_B)],
                    partial_hbm.at[cid, 1, pl.ds(sid * ROWS_B, ROWS_B)])

    gstart = jnp.where(w < 15, 3 * w, 45 + 2 * (w - 15))
    gcnt = jnp.where(w < 15, 3, jnp.where(w < 31, 2, 1))

    def gbody(s3, _):
        r = gstart + s3
        pltpu.sync_copy(onodes_hbm.at[pl.ds(r * 128, 128)], oidx_v.at[s3])
        pltpu.async_copy(nemb_hbm.at[oidx_v.at[s3]], grows_v, sem).wait()
        pltpu.sync_copy(grows_v, gathered_hbm.at[pl.ds(r * 128, 128)])
        return 0

    lax.fori_loop(0, gcnt, gbody, 0)

    izero = jnp.zeros((16,), jnp.int32)

    def pbody(_, __):
        for l in range(8):
            oidx_v[1, pl.ds(l * 16, 16)] = izero
        pltpu.sync_copy(onodes_hbm.at[pl.ds(G_FULL * 128, B - G_FULL * 128)],
                        oidx_v.at[1, pl.ds(0, B - G_FULL * 128)])
        pltpu.async_copy(nemb_hbm.at[oidx_v.at[1]], grows_v, sem).wait()
        pltpu.sync_copy(grows_v, gathered_hbm.at[pl.ds(G_FULL * 128, 128)])
        return 0

    lax.fori_loop(0, jnp.where(w == 31, 1, 0), pbody, 0)


def _tc_body(partial_ref, gathered_ref, s1_ref, s2_ref, w_ref, out_ref):
    p = partial_ref[...]
    agg0 = p[0, 0] + p[1, 0]
    agg1 = p[0, 1] + p[1, 1]
    s1 = s1_ref[...]
    s2 = s2_ref[...]
    h0 = jnp.tanh(jnp.dot(agg0, s1[0], preferred_element_type=jnp.float32))
    h1 = jnp.tanh(jnp.dot(agg1, s1[1], preferred_element_type=jnp.float32))
    l0 = jnp.sum(h0 * s2[0][None, :], axis=1, keepdims=True)
    l1 = jnp.sum(h1 * s2[1][None, :], axis=1, keepdims=True)
    m = jnp.maximum(l0, l1)
    e0 = jnp.exp(l0 - m)
    e1 = jnp.exp(l1 - m)
    inv = 1.0 / (e0 + e1)
    mixed = (e0 * inv) * agg0 + (e1 * inv) * agg1
    g = gathered_ref[...]
    wts = w_ref[...]
    for t in range(T):
        v = g + jnp.dot(mixed, wts[t], preferred_element_type=jnp.float32)
        nrm = jnp.sqrt(jnp.sum(v * v, axis=1, keepdims=True))
        out_ref[:, t, :] = v / jnp.maximum(nrm, 1e-12)


def kernel(node_embeddings, node_type_embeddings, trans_weights,
           trans_weights_s1, trans_weights_s2, input_nodes, output_nodes,
           edge_src, edge_dst):
    nte = node_type_embeddings.reshape(N_NODES * T, U)

    src_p = edge_src.reshape(T, ROWS_T, 128)
    dst_p = edge_dst.reshape(T, ROWS_T, 128)

    sc = functools.partial(
        pl.kernel,
        out_type=[
            jax.ShapeDtypeStruct((NC, T, B_ACC, U), jnp.float32),
            jax.ShapeDtypeStruct((BG, D), jnp.float32),
        ],
        mesh=plsc.VectorSubcoreMesh(core_axis_name="c", subcore_axis_name="s"),
        compiler_params=pltpu.CompilerParams(use_tc_tiling_on_sc=False),
        scratch_types=[
            pltpu.VMEM((RC, 128), jnp.int32),
            pltpu.VMEM((RC, 128), jnp.int32),
            pltpu.VMEM((RC, 128), jnp.int32),
            pltpu.VMEM((RC, 128, U), jnp.float32),
            pltpu.VMEM((ROWS_B, U), jnp.float32),
            pltpu.VMEM((3, 128), jnp.int32),
            pltpu.VMEM((128, D), jnp.float32),
            pltpu.VMEM_SHARED((B_ACC, U), jnp.float32),
            pltpu.VMEM_SHARED((B_ACC, U), jnp.float32),
            pltpu.SemaphoreType.DMA,
        ],
    )(_sc_kernel)
    partial, gathered = sc(src_p, dst_p, nte, output_nodes, node_embeddings)

    BB = 2000
    s2r = trans_weights_s2.reshape(T, 16)
    out = pl.pallas_call(
        _tc_body,
        grid=(B // BB,),
        in_specs=[
            pl.BlockSpec((NC, T, BB, U), lambda i: (0, 0, i, 0)),
            pl.BlockSpec((BB, D), lambda i: (i, 0)),
            pl.BlockSpec((T, U, 16), lambda i: (0, 0, 0)),
            pl.BlockSpec((T, 16), lambda i: (0, 0)),
            pl.BlockSpec((T, U, D), lambda i: (0, 0, 0)),
        ],
        out_specs=pl.BlockSpec((BB, T, D), lambda i: (i, 0, 0)),
        out_shape=jax.ShapeDtypeStruct((B, T, D), jnp.float32),
    )(partial, gathered, trans_weights_s1, s2r, trans_weights)
    return out


# speedup vs baseline: 1.9041x; 1.0459x over previous
import functools

import jax
import jax.numpy as jnp
from jax import lax
from jax.experimental import pallas as pl
from jax.experimental.pallas import tpu as pltpu
from jax.experimental.pallas import tpu_sc as plsc

N_NODES = 100000
T = 2
U = 16
D = 128
B = 10000
E = 1600000

NC = 2
NS = 16
NW = NC * NS

ROWS_T = 12544
RC = 16
CHUNKS_T = ROWS_T // RC
CPW_HI = 25
CPW_LO = 24

B_ACC = 10240
ROWS_B = B_ACC // NS
G_FULL = B // 128
G_ROWS = G_FULL + 1
BG = G_ROWS * 128


def _sc_kernel(src_hbm, dst_hbm, nte_hbm, onodes_hbm, nemb_hbm,
               partial_hbm, gathered_hbm,
               src_v, dst_v, gidx_v, rows_v, zbuf_v, oidx_v, grows_v,
               acc0, acc1, sem):
    cid = lax.axis_index("c")
    sid = lax.axis_index("s")
    w = cid * NS + sid

    zero = jnp.zeros((16,), jnp.float32)

    def zbody(i, _):
        zbuf_v[i, :] = zero
        return 0

    lax.fori_loop(0, ROWS_B, zbody, 0)
    pltpu.sync_copy(zbuf_v, acc0.at[pl.ds(sid * ROWS_B, ROWS_B)])
    pltpu.sync_copy(zbuf_v, acc1.at[pl.ds(sid * ROWS_B, ROWS_B)])
    plsc.subcore_barrier()

    nchunks = jnp.where(w < 16, CPW_HI, CPW_LO)
    chunk0 = jnp.where(w < 16, w * CPW_HI,
                       16 * CPW_HI + (w - 16) * CPW_LO)

    for t in range(T):
        acc = acc0 if t == 0 else acc1

        def cbody(c, _, t=t, acc=acc):
            base = (chunk0 + c) * RC
            pltpu.sync_copy(src_hbm.at[t, pl.ds(base, RC)], src_v)
            pltpu.sync_copy(dst_hbm.at[t, pl.ds(base, RC)], dst_v)

            def ibody(j, _):
                for l in range(8):
                    gidx_v[j, pl.ds(l * 16, 16)] = (
                        src_v[j, pl.ds(l * 16, 16)] * T + t)
                return 0

            lax.fori_loop(0, RC, ibody, 0)

            def gfire(j, _):
                pltpu.make_async_copy(
                    nte_hbm.at[gidx_v.at[j]], rows_v.at[j], sem).start()
                return 0

            lax.fori_loop(0, RC, gfire, 0)

            def gdrain(j, _):
                pltpu.make_async_copy(
                    nte_hbm.at[gidx_v.at[j]], rows_v.at[j], sem).wait()
                return 0

            lax.fori_loop(0, RC, gdrain, 0)

            def sbody(j, _):
                pltpu.sync_copy(rows_v.at[j], acc.at[dst_v.at[j]], add=True)
                return 0

            lax.fori_loop(0, RC, sbody, 0)
            return 0

        lax.fori_loop(0, nchunks, cbody, 0)

    plsc.subcore_barrier()

    pltpu.sync_copy(acc0.at[pl.ds(sid * ROWS_B, ROWS_B)],
                    partial_hbm.at[cid, 0, pl.ds(sid * ROWS_B, ROWS_B)])
    pltpu.sync_copy(acc1.at[pl.ds(sid * ROWS_B, ROWS_B)],
                    partial_hbm.at[cid, 1, pl.ds(sid * ROWS_B, ROWS_B)])

    gstart = jnp.where(w < 15, 3 * w, 45 + 2 * (w - 15))
    gcnt = jnp.where(w < 15, 3, jnp.where(w < 31, 2, 1))

    def gbody(s3, _):
        r = gstart + s3
        pltpu.sync_copy(onodes_hbm.at[pl.ds(r * 128, 128)], oidx_v.at[s3])
        pltpu.async_copy(nemb_hbm.at[oidx_v.at[s3]], grows_v, sem).wait()
        pltpu.sync_copy(grows_v, gathered_hbm.at[pl.ds(r * 128, 128)])
        return 0

    lax.fori_loop(0, gcnt, gbody, 0)

    izero = jnp.zeros((16,), jnp.int32)

    def pbody(_, __):
        for l in range(8):
            oidx_v[1, pl.ds(l * 16, 16)] = izero
        pltpu.sync_copy(onodes_hbm.at[pl.ds(G_FULL * 128, B - G_FULL * 128)],
                        oidx_v.at[1, pl.ds(0, B - G_FULL * 128)])
        pltpu.async_copy(nemb_hbm.at[oidx_v.at[1]], grows_v, sem).wait()
        pltpu.sync_copy(grows_v, gathered_hbm.at[pl.ds(G_FULL * 128, 128)])
        return 0

    lax.fori_loop(0, jnp.where(w == 31, 1, 0), pbody, 0)


def _tc_body(partial_ref, gathered_ref, s1_ref, s2_ref, w_ref, out_ref):
    p = partial_ref[...]
    agg0 = p[0, 0] + p[1, 0]
    agg1 = p[0, 1] + p[1, 1]
    s1 = s1_ref[...]
    s2 = s2_ref[...]
    h0 = jnp.tanh(jnp.dot(agg0, s1[0], preferred_element_type=jnp.float32))
    h1 = jnp.tanh(jnp.dot(agg1, s1[1], preferred_element_type=jnp.float32))
    l0 = jnp.sum(h0 * s2[0][None, :], axis=1, keepdims=True)
    l1 = jnp.sum(h1 * s2[1][None, :], axis=1, keepdims=True)
    m = jnp.maximum(l0, l1)
    e0 = jnp.exp(l0 - m)
    e1 = jnp.exp(l1 - m)
    inv = 1.0 / (e0 + e1)
    mixed = (e0 * inv) * agg0 + (e1 * inv) * agg1
    g = gathered_ref[...]
    wts = w_ref[...]
    for t in range(T):
        v = g + jnp.dot(mixed, wts[t], preferred_element_type=jnp.float32)
        nrm = jnp.sqrt(jnp.sum(v * v, axis=1, keepdims=True))
        out_ref[:, t, :] = v / jnp.maximum(nrm, 1e-12)


def kernel(node_embeddings, node_type_embeddings, trans_weights,
           trans_weights_s1, trans_weights_s2, input_nodes, output_nodes,
           edge_src, edge_dst):
    nte = node_type_embeddings.reshape(N_NODES * T, U)

    pad = ROWS_T * 128 - E
    pad_src = jnp.arange(pad, dtype=jnp.int32) % 128
    pad_dst = B + (jnp.arange(pad, dtype=jnp.int32) % 128)
    src_p = jnp.concatenate(
        [edge_src, jnp.broadcast_to(pad_src, (T, pad))], axis=1
    ).reshape(T, ROWS_T, 128)
    dst_p = jnp.concatenate(
        [edge_dst, jnp.broadcast_to(pad_dst, (T, pad))], axis=1
    ).reshape(T, ROWS_T, 128)

    sc = functools.partial(
        pl.kernel,
        out_type=[
            jax.ShapeDtypeStruct((NC, T, B_ACC, U), jnp.float32),
            jax.ShapeDtypeStruct((BG, D), jnp.float32),
        ],
        mesh=plsc.VectorSubcoreMesh(core_axis_name="c", subcore_axis_name="s"),
        compiler_params=pltpu.CompilerParams(use_tc_tiling_on_sc=False),
        scratch_types=[
            pltpu.VMEM((RC, 128), jnp.int32),
            pltpu.VMEM((RC, 128), jnp.int32),
            pltpu.VMEM((RC, 128), jnp.int32),
            pltpu.VMEM((RC, 128, U), jnp.float32),
            pltpu.VMEM((ROWS_B, U), jnp.float32),
            pltpu.VMEM((3, 128), jnp.int32),
            pltpu.VMEM((128, D), jnp.float32),
            pltpu.VMEM_SHARED((B_ACC, U), jnp.float32),
            pltpu.VMEM_SHARED((B_ACC, U), jnp.float32),
            pltpu.SemaphoreType.DMA,
        ],
    )(_sc_kernel)
    partial, gathered = sc(src_p, dst_p, nte, output_nodes, node_embeddings)

    BB = 2000
    s2r = trans_weights_s2.reshape(T, 16)
    out = pl.pallas_call(
        _tc_body,
        grid=(B // BB,),
        in_specs=[
            pl.BlockSpec((NC, T, BB, U), lambda i: (0, 0, i, 0)),
            pl.BlockSpec((BB, D), lambda i: (i, 0)),
            pl.BlockSpec((T, U, 16), lambda i: (0, 0, 0)),
            pl.BlockSpec((T, 16), lambda i: (0, 0)),
            pl.BlockSpec((T, U, D), lambda i: (0, 0, 0)),
        ],
        out_specs=pl.BlockSpec((BB, T, D), lambda i: (i, 0, 0)),
        out_shape=jax.ShapeDtypeStruct((B, T, D), jnp.float32),
    )(partial, gathered, trans_weights_s1, s2r, trans_weights)
    return out
